# baseline (speedup 1.0000x reference)
import functools

import jax
import jax.numpy as jnp
import numpy as np
from jax import lax
from jax.experimental import pallas as pl
from jax.experimental.pallas import tpu as pltpu
from jax.experimental.pallas import tpu_sc as plsc

N_NODES = 10000
D_FEAT = 128
N_EDGES = 320000

NC = 2
NS = 16
NW = NC * NS
EPW = N_EDGES // NW
K = 80
NB = 5
CB = 25
N_PAD = 10240
RPT = N_PAD // NS

_SIGMA = np.empty(D_FEAT, dtype=np.int32)
for _c in range(D_FEAT // 32):
    for _j in range(16):
        _SIGMA[32 * _c + _j] = 32 * _c + 2 * _j
        _SIGMA[32 * _c + 16 + _j] = 32 * _c + 2 * _j + 1
_COL_PERM = np.argsort(_SIGMA)


def _sc_scatter_body(x_hbm, src_hbm, dst_hbm, ew_hbm, acc_hbm,
                     acc_sh, src_v, dst_v, ew_v,
                     bf_a, bf_b, z_a, z_b,
                     ga, gb, sa, sb):
    c = lax.axis_index("c")
    s = lax.axis_index("s")
    gid = c * NS + s

    def _zero(i, carry):
        z_a[i // 8, pl.ds((i % 8) * 16, 16)] = jnp.zeros((16,), jnp.float32)
        return carry
    lax.fori_loop(0, K * 8, _zero, 0)
    for j in range(RPT // K):
        pltpu.sync_copy(z_a, acc_sh.at[pl.ds(s * RPT + j * K, K)])
    plsc.subcore_barrier()

    def _scale(bf_buf, z_buf, base, g, inner):
        w_win = ew_v[pl.ds(base + g * 16, 16)]
        for r16 in range(16):
            r = g * 16 + r16
            w16 = jnp.broadcast_to(w_win[r16], (16,))
            for cc in range(4):
                v = bf_buf[r, pl.ds(cc * 16, 16)]
                lo = plsc.bitcast(lax.shift_left(v, 16), jnp.float32)
                hi = plsc.bitcast(
                    jnp.bitwise_and(v, jnp.int32(-65536)), jnp.float32)
                z_buf[r, pl.ds(cc * 32, 16)] = lo * w16
                z_buf[r, pl.ds(cc * 32 + 16, 16)] = hi * w16
        return inner

    def _widen_scale(bf_buf, z_buf, k):
        lax.fori_loop(0, K // 16,
                      functools.partial(_scale, bf_buf, z_buf, k * K), 0)

    def _g_issue(buf, sem, k):
        pltpu.async_copy(x_hbm.at[src_v.at[k]], buf, sem)

    def _g_wait(buf, sem, k):
        pltpu.make_async_copy(x_hbm.at[src_v.at[k]], buf, sem).wait()

    def _s_issue(zbuf, sem, k):
        pltpu.async_copy(zbuf, acc_sh.at[dst_v.at[k]], sem, add=True)

    def _s_wait(zbuf, sem, k):
        pltpu.make_async_copy(zbuf, acc_sh.at[dst_v.at[k]], sem).wait()

    def _block(b, carry):
        pltpu.sync_copy(src_hbm.at[gid, b], src_v)
        pltpu.sync_copy(dst_hbm.at[gid, b], dst_v)
        pltpu.sync_copy(ew_hbm.at[gid, b], ew_v)

        _g_issue(bf_a, ga, 0)
        _g_issue(bf_b, gb, 1)

        def _pair(p, c2):
            k0 = 2 * p
            _g_wait(bf_a, ga, k0)

            @pl.when(p > 0)
            def _():
                _s_wait(z_a, sa, k0 - 2)
            _widen_scale(bf_a, z_a, k0)
            _s_issue(z_a, sa, k0)
            _g_issue(bf_a, ga, k0 + 2)

            _g_wait(bf_b, gb, k0 + 1)

            @pl.when(p > 0)
            def _():
                _s_wait(z_b, sb, k0 - 1)
            _widen_scale(bf_b, z_b, k0 + 1)
            _s_issue(z_b, sb, k0 + 1)

            @pl.when(p < CB // 2 - 1)
            def _():
                _g_issue(bf_b, gb, k0 + 3)
            return c2
        lax.fori_loop(0, CB // 2, _pair, 0)

        kt = CB - 1
        _g_wait(bf_a, ga, kt)
        _s_wait(z_a, sa, kt - 2)
        _widen_scale(bf_a, z_a, kt)
        _s_issue(z_a, sa, kt)
        _s_wait(z_b, sb, kt - 1)
        _s_wait(z_a, sa, kt)
        return carry
    lax.fori_loop(0, NB, _block, 0)

    plsc.subcore_barrier()
    pltpu.sync_copy(acc_sh.at[pl.ds(s * RPT, RPT)],
                    acc_hbm.at[c, pl.ds(s * RPT, RPT)])


_sc_scatter = functools.partial(
    pl.kernel,
    out_type=jax.ShapeDtypeStruct((NC, N_PAD, D_FEAT), jnp.float32),
    mesh=plsc.VectorSubcoreMesh(core_axis_name="c", subcore_axis_name="s"),
    compiler_params=pltpu.CompilerParams(needs_layout_passes=False,
                                         use_tc_tiling_on_sc=False),
    scratch_types=[
        pltpu.VMEM_SHARED((N_PAD, D_FEAT), jnp.float32),
        pltpu.VMEM((CB, K), jnp.int32),
        pltpu.VMEM((CB, K), jnp.int32),
        pltpu.VMEM((CB * K,), jnp.float32),
        pltpu.VMEM((K, D_FEAT // 2), jnp.int32),
        pltpu.VMEM((K, D_FEAT // 2), jnp.int32),
        pltpu.VMEM((K, D_FEAT), jnp.float32),
        pltpu.VMEM((K, D_FEAT), jnp.float32),
        pltpu.SemaphoreType.DMA,
        pltpu.SemaphoreType.DMA,
        pltpu.SemaphoreType.DMA,
        pltpu.SemaphoreType.DMA,
    ],
)(_sc_scatter_body)


def _finish_body(acc_ref, o_ref):
    t = acc_ref[0] + acc_ref[1]
    t = jnp.maximum(t, 0.0)
    nrm = jnp.sqrt(jnp.sum(t * t, axis=1, keepdims=True))
    o_ref[...] = t / jnp.maximum(nrm, 1e-12)


_ROWS_PER_BLK = 1024


def _finish(acc):
    return pl.pallas_call(
        _finish_body,
        grid=(N_PAD // _ROWS_PER_BLK,),
        in_specs=[pl.BlockSpec((NC, _ROWS_PER_BLK, D_FEAT),
                               lambda i: (0, i, 0))],
        out_specs=pl.BlockSpec((_ROWS_PER_BLK, D_FEAT), lambda i: (i, 0)),
        out_shape=jax.ShapeDtypeStruct((N_PAD, D_FEAT), jnp.float32),
    )(acc)


def kernel(x, edge, edge_weight):
    xb = x[:, _COL_PERM].astype(jnp.bfloat16)
    xp = lax.bitcast_convert_type(
        xb.reshape(N_NODES, D_FEAT // 2, 2), jnp.int32)
    src = edge[0].reshape(NW, NB, CB, K)
    dst = edge[2].reshape(NW, NB, CB, K)
    ew = edge_weight.reshape(NW, NB, CB * K)
    acc = _sc_scatter(xp, src, dst, ew)
    return _finish(acc)[:N_NODES]
